# Initial kernel scaffold; baseline (speedup 1.0000x reference)
import jax
import jax.numpy as jnp
from jax.experimental import pallas as pl


def kernel(rms_vel):
    raise NotImplementedError("write your pallas kernel here")


# speedup vs baseline: 8.3415x; 8.3415x over previous
import functools

import numpy as np
import jax
import jax.numpy as jnp
from jax import lax
from jax.experimental import pallas as pl
from jax.experimental.pallas import tpu as pltpu
from jax.experimental.pallas import tpu_sc as plsc

DT = 0.001
DZ = 10.0
NZ = 70
VMIN = 1200.0
VMAX = 6000.0
LAM = 10.0
EPS = 1e-06

NT = 1024
NXB = 256

NW = 32
NL = 16
NZP = 80
NGRP = NZP // NL


def _build_mats():
    n, lam = NT, LAM
    m = np.zeros((n, n), dtype=np.float64)
    i = np.arange(n)
    m[i, i] = 1.0 + 2.0 * lam
    m[0, 0] = 1.0 + lam
    m[n - 1, n - 1] = 1.0 + lam
    m[i[1:], i[:-1]] = -lam
    m[i[:-1], i[1:]] = -lam
    a = np.linalg.inv(m)
    c = 0.5 * DT * np.cumsum(a, axis=0)
    return np.asarray(a, np.float32), np.asarray(c, np.float32)


_A_NP, _C_NP = _build_mats()


def _tc_body(rms_ref, a_ref, c_ref, sm_ref, smt_ref, zt_ref):
    v = rms_ref[0, 0]
    t = lax.broadcasted_iota(jnp.int32, (NT, NXB), 0).astype(jnp.float32) * DT
    y = v * v * t
    dy = y[1:, :] - y[:-1, :]
    vint2_tail = dy / DT
    v0sq = jnp.clip(v[0:1, :] * v[0:1, :], VMIN * VMIN, VMAX * VMAX)
    vint2 = jnp.concatenate([v0sq, vint2_tail], axis=0)
    vint2 = jnp.clip(vint2, VMIN * VMIN, VMAX * VMAX)
    tv = jnp.sqrt(vint2 + EPS)

    sm = jnp.dot(a_ref[...], tv, preferred_element_type=jnp.float32)
    sm_ref[0, 0] = sm
    dn = (((0,), (0,)), ((), ()))
    smt_ref[0] = lax.dot_general(tv, a_ref[...], dn,
                                 preferred_element_type=jnp.float32)
    dnc = (((0,), (1,)), ((), ()))
    zt_ref[0] = lax.dot_general(tv, c_ref[...], dnc,
                                precision=lax.Precision.HIGHEST,
                                preferred_element_type=jnp.float32)


def _tc_call(rms_vel, a_mat, c_mat):
    b, _, nt, nx = rms_vel.shape
    return pl.pallas_call(
        _tc_body,
        grid=(b, nx // NXB),
        in_specs=[
            pl.BlockSpec((1, 1, NT, NXB), lambda i, j: (i, 0, 0, j)),
            pl.BlockSpec((NT, NT), lambda i, j: (0, 0)),
            pl.BlockSpec((NT, NT), lambda i, j: (0, 0)),
        ],
        out_specs=[
            pl.BlockSpec((1, 1, NT, NXB), lambda i, j: (i, 0, 0, j)),
            pl.BlockSpec((1, NXB, NT), lambda i, j: (i, j, 0)),
            pl.BlockSpec((1, NXB, NT), lambda i, j: (i, j, 0)),
        ],
        out_shape=[
            jax.ShapeDtypeStruct((b, 1, nt, nx), jnp.float32),
            jax.ShapeDtypeStruct((b, nx, nt), jnp.float32),
            jax.ShapeDtypeStruct((b, nx, nt), jnp.float32),
        ],
    )(rms_vel, a_mat, c_mat)


def _sc_body(z_hbm, v_hbm, out_hbm, z_v, v_v, o_v):
    ntr = z_hbm.shape[0]
    per_w = ntr // NW
    wid = lax.axis_index("s") * 2 + lax.axis_index("c")
    base = wid * per_w

    def trace_body(i, carry):
        tr = base + i
        pltpu.sync_copy(z_hbm.at[tr], z_v)
        pltpu.sync_copy(v_hbm.at[tr], v_v)
        for g in range(NGRP):
            kq = lax.iota(jnp.int32, NL) + (g * NL)
            qf = kq.astype(jnp.float32) * DZ
            lo = jnp.full((NL,), -1, jnp.int32)
            hi = jnp.full((NL,), NT, jnp.int32)
            for _ in range(11):
                mid = jnp.maximum((lo + hi) >> 1, 0)
                zm = plsc.load_gather(z_v, [mid])
                pred = zm < qf
                lo = jnp.where(pred, mid, lo)
                hi = jnp.where(pred, hi, mid)
            idx1 = jnp.minimum(hi, NT - 1)
            idx0 = jnp.maximum(idx1 - 1, 0)
            z0 = plsc.load_gather(z_v, [idx0])
            z1 = plsc.load_gather(z_v, [idx1])
            v0 = plsc.load_gather(v_v, [idx0])
            v1 = plsc.load_gather(v_v, [idx1])
            denom = jnp.maximum(z1 - z0, EPS)
            w = jnp.clip((qf - z0) / denom, 0.0, 1.0)
            o_v[pl.ds(g * NL, NL)] = v0 + w * (v1 - v0)
        pltpu.sync_copy(o_v, out_hbm.at[tr])
        return carry

    lax.fori_loop(0, per_w, trace_body, 0)


def _depth_resample(zflat, vflat):
    ntr = zflat.shape[0]
    mesh = plsc.VectorSubcoreMesh(core_axis_name="c", subcore_axis_name="s")
    fn = pl.kernel(
        _sc_body,
        mesh=mesh,
        out_type=jax.ShapeDtypeStruct((ntr, NZP), jnp.float32),
        scratch_types=[
            pltpu.VMEM((NT,), jnp.float32),
            pltpu.VMEM((NT,), jnp.float32),
            pltpu.VMEM((NZP,), jnp.float32),
        ],
        compiler_params=pltpu.CompilerParams(needs_layout_passes=False),
    )
    return fn(zflat, vflat)


def kernel(rms_vel):
    b, _, nt, nx = rms_vel.shape
    a_mat = jnp.asarray(_A_NP)
    c_mat = jnp.asarray(_C_NP)
    sm, smt, zt = _tc_call(rms_vel, a_mat, c_mat)
    depth_flat = _depth_resample(zt.reshape(b * nx, nt),
                                 smt.reshape(b * nx, nt))
    depth = depth_flat[:, :NZ].reshape(b, nx, NZ)
    depth_vel = jnp.transpose(depth, (0, 2, 1))[:, None]
    return depth_vel, sm


# speedup vs baseline: 16.1592x; 1.9372x over previous
import functools

import numpy as np
import jax
import jax.numpy as jnp
from jax import lax
from jax.experimental import pallas as pl
from jax.experimental.pallas import tpu as pltpu
from jax.experimental.pallas import tpu_sc as plsc

DT = 0.001
DZ = 10.0
NZ = 70
VMIN = 1200.0
VMAX = 6000.0
LAM = 10.0
EPS = 1e-06

NT = 1024
NXB = 256

NW = 32
NL = 16
NZP = 80
NGRP = NZP // NL
CH = 16


def _build_mats():
    n, lam = NT, LAM
    m = np.zeros((n, n), dtype=np.float64)
    i = np.arange(n)
    m[i, i] = 1.0 + 2.0 * lam
    m[0, 0] = 1.0 + lam
    m[n - 1, n - 1] = 1.0 + lam
    m[i[1:], i[:-1]] = -lam
    m[i[:-1], i[1:]] = -lam
    a = np.linalg.inv(m)
    c = 0.5 * DT * np.cumsum(a, axis=0)
    return np.asarray(a, np.float32), np.asarray(c, np.float32)


_A_NP, _C_NP = _build_mats()


def _bsearch_bounds():
    los, his, iters = [], [], []
    for g in range(NGRP):
        ks = np.minimum(np.arange(g * NL, (g + 1) * NL), NZ - 1)
        q = ks * 10
        lo = np.maximum(-1, (q * 100) // 302 - 2).astype(np.int32)
        hi = np.minimum(NT, (q * 100) // 58 + 3).astype(np.int32)
        width = int(np.max(hi - lo))
        it = 0
        while (1 << it) < width:
            it += 1
        los.append(lo)
        his.append(hi)
        iters.append(it)
    return los, his, iters


_BS_LO, _BS_HI, _BS_ITERS = _bsearch_bounds()


def _tc_body(rms_ref, a_ref, c_ref, sm_ref, zvt_ref):
    v = rms_ref[0, 0]
    t = lax.broadcasted_iota(jnp.int32, (NT, NXB), 0).astype(jnp.float32) * DT
    y = v * v * t
    dy = y[1:, :] - y[:-1, :]
    vint2_tail = dy / DT
    v0sq = jnp.clip(v[0:1, :] * v[0:1, :], VMIN * VMIN, VMAX * VMAX)
    vint2 = jnp.concatenate([v0sq, vint2_tail], axis=0)
    vint2 = jnp.clip(vint2, VMIN * VMIN, VMAX * VMAX)
    tv = jnp.sqrt(vint2 + EPS)

    sm = jnp.dot(a_ref[...], tv, preferred_element_type=jnp.float32)
    sm_ref[0, 0] = sm
    dn0 = (((0,), (0,)), ((), ()))
    zvt_ref[0, :, NT:] = lax.dot_general(tv, a_ref[...], dn0,
                                         preferred_element_type=jnp.float32)
    dn1 = (((0,), (1,)), ((), ()))
    zvt_ref[0, :, :NT] = lax.dot_general(tv, c_ref[...], dn1,
                                         precision=lax.Precision.HIGHEST,
                                         preferred_element_type=jnp.float32)


def _tc_call(rms_vel, a_mat, c_mat):
    b, _, nt, nx = rms_vel.shape
    return pl.pallas_call(
        _tc_body,
        grid=(b, nx // NXB),
        in_specs=[
            pl.BlockSpec((1, 1, NT, NXB), lambda i, j: (i, 0, 0, j)),
            pl.BlockSpec((NT, NT), lambda i, j: (0, 0)),
            pl.BlockSpec((NT, NT), lambda i, j: (0, 0)),
        ],
        out_specs=[
            pl.BlockSpec((1, 1, NT, NXB), lambda i, j: (i, 0, 0, j)),
            pl.BlockSpec((1, NXB, 2 * NT), lambda i, j: (i, j, 0)),
        ],
        out_shape=[
            jax.ShapeDtypeStruct((b, 1, nt, nx), jnp.float32),
            jax.ShapeDtypeStruct((b, nx, 2 * nt), jnp.float32),
        ],
    )(rms_vel, a_mat, c_mat)


def _search_chunk(zv_buf, o_buf, out_hbm, row):
    qfs, los, his = [], [], []
    for g in range(NGRP):
        kq = jnp.minimum(lax.iota(jnp.int32, NL) + (g * NL), NZ - 1)
        qfs.append(kq.astype(jnp.float32) * DZ)
        q100 = kq * 1000
        los.append(jnp.maximum(q100 // 302 - 2, -1))
        his.append(jnp.minimum(q100 // 58 + 3, NT))

    def trace_body(t, carry):
        ts = jnp.full((NL,), t, jnp.int32)
        for g in range(NGRP):
            qf = qfs[g]
            lo = los[g]
            hi = his[g]
            for _ in range(_BS_ITERS[g]):
                mid = jnp.maximum((lo + hi) >> 1, 0)
                zm = plsc.load_gather(zv_buf, [ts, mid])
                pred = zm < qf
                lo = jnp.where(pred, mid, lo)
                hi = jnp.where(pred, hi, mid)
            idx1 = jnp.minimum(hi, NT - 1)
            idx0 = jnp.maximum(idx1 - 1, 0)
            z0 = plsc.load_gather(zv_buf, [ts, idx0])
            z1 = plsc.load_gather(zv_buf, [ts, idx1])
            v0 = plsc.load_gather(zv_buf, [ts, idx0 + NT])
            v1 = plsc.load_gather(zv_buf, [ts, idx1 + NT])
            denom = jnp.maximum(z1 - z0, EPS)
            w = jnp.clip((qf - z0) / denom, 0.0, 1.0)
            o_buf[t, pl.ds(g * NL, NL)] = v0 + w * (v1 - v0)
        return carry

    lax.fori_loop(0, CH, trace_body, 0)
    pltpu.sync_copy(o_buf, out_hbm.at[pl.ds(row, CH)])


def _sc_body(zv_hbm, out_hbm, zv0, zv1, o_buf, sem0, sem1):
    ntr = zv_hbm.shape[0]
    per_w = ntr // NW
    nch = per_w // CH
    wid = lax.axis_index("s") * 2 + lax.axis_index("c")
    base = wid * per_w

    def issue(row, buf, sem):
        pltpu.async_copy(zv_hbm.at[pl.ds(row, CH)], buf, sem)

    def drain(buf, sem):
        pltpu.make_async_copy(zv_hbm.at[pl.ds(0, CH)], buf, sem).wait()

    issue(base, zv0, sem0)

    def pair_body(i, carry):
        r0 = base + (2 * i) * CH
        r1 = r0 + CH
        issue(r1, zv1, sem1)
        drain(zv0, sem0)
        _search_chunk(zv0, o_buf, out_hbm, r0)

        @pl.when(i < (nch // 2) - 1)
        def _():
            issue(r1 + CH, zv0, sem0)

        drain(zv1, sem1)
        _search_chunk(zv1, o_buf, out_hbm, r1)
        return carry

    lax.fori_loop(0, nch // 2, pair_body, 0)


def _depth_resample(zvflat):
    ntr = zvflat.shape[0]
    mesh = plsc.VectorSubcoreMesh(core_axis_name="c", subcore_axis_name="s")
    fn = pl.kernel(
        _sc_body,
        mesh=mesh,
        out_type=jax.ShapeDtypeStruct((ntr, NZP), jnp.float32),
        scratch_types=[
            pltpu.VMEM((CH, 2 * NT), jnp.float32),
            pltpu.VMEM((CH, 2 * NT), jnp.float32),
            pltpu.VMEM((CH, NZP), jnp.float32),
            pltpu.SemaphoreType.DMA,
            pltpu.SemaphoreType.DMA,
        ],
        compiler_params=pltpu.CompilerParams(needs_layout_passes=False),
    )
    return fn(zvflat)


def kernel(rms_vel):
    b, _, nt, nx = rms_vel.shape
    a_mat = jnp.asarray(_A_NP)
    c_mat = jnp.asarray(_C_NP)
    sm, zvt = _tc_call(rms_vel, a_mat, c_mat)
    depth_flat = _depth_resample(zvt.reshape(b * nx, 2 * nt))
    depth = depth_flat[:, :NZ].reshape(b, nx, NZ)
    depth_vel = jnp.transpose(depth, (0, 2, 1))[:, None]
    return depth_vel, sm


# speedup vs baseline: 20.6677x; 1.2790x over previous
import functools

import numpy as np
import jax
import jax.numpy as jnp
from jax import lax
from jax.experimental import pallas as pl
from jax.experimental.pallas import tpu as pltpu
from jax.experimental.pallas import tpu_sc as plsc

DT = 0.001
DZ = 10.0
NZ = 70
VMIN = 1200.0
VMAX = 6000.0
LAM = 10.0
EPS = 1e-06

NT = 1024
NXB = 256

NW = 32
NL = 16
NZP = 80
NGRP = NZP // NL
CH = 16


def _build_mats():
    n, lam = NT, LAM
    m = np.zeros((n, n), dtype=np.float64)
    i = np.arange(n)
    m[i, i] = 1.0 + 2.0 * lam
    m[0, 0] = 1.0 + lam
    m[n - 1, n - 1] = 1.0 + lam
    m[i[1:], i[:-1]] = -lam
    m[i[:-1], i[1:]] = -lam
    a = np.linalg.inv(m)
    c = 0.5 * DT * np.cumsum(a, axis=0)
    ch = c.astype(np.float32).astype(jnp.bfloat16)
    cl = (c.astype(np.float32) - np.asarray(ch, np.float32)).astype(jnp.bfloat16)
    return np.asarray(a, np.float32), ch, cl


_A_NP, _CH_NP, _CL_NP = _build_mats()


def _bsearch_bounds():
    los, his, iters = [], [], []
    for g in range(NGRP):
        ks = np.minimum(np.arange(g * NL, (g + 1) * NL), NZ - 1)
        q = ks * 10
        lo = np.maximum(-1, (q * 100) // 302 - 2).astype(np.int32)
        hi = np.minimum(NT, (q * 100) // 58 + 3).astype(np.int32)
        width = int(np.max(hi - lo))
        it = 0
        while (1 << it) < width:
            it += 1
        los.append(lo)
        his.append(hi)
        iters.append(it)
    return los, his, iters


_BS_LO, _BS_HI, _BS_ITERS = _bsearch_bounds()


def _tc_body(rms_ref, a_ref, ch_ref, cl_ref, sm_ref, zvt_ref):
    v = rms_ref[0, 0]
    t = lax.broadcasted_iota(jnp.int32, (NT, NXB), 0).astype(jnp.float32) * DT
    y = v * v * t
    dy = y[1:, :] - y[:-1, :]
    vint2_tail = dy / DT
    v0sq = jnp.clip(v[0:1, :] * v[0:1, :], VMIN * VMIN, VMAX * VMAX)
    vint2 = jnp.concatenate([v0sq, vint2_tail], axis=0)
    vint2 = jnp.clip(vint2, VMIN * VMIN, VMAX * VMAX)
    tv = jnp.sqrt(vint2 + EPS)

    sm = jnp.dot(a_ref[...], tv, preferred_element_type=jnp.float32)
    sm_ref[0, 0] = sm
    zvt_ref[0, :, NT:] = sm.T
    th = tv.astype(jnp.bfloat16)
    tl = (tv - th.astype(jnp.float32)).astype(jnp.bfloat16)
    zcol = (jnp.dot(ch_ref[...], th, preferred_element_type=jnp.float32)
            + jnp.dot(ch_ref[...], tl, preferred_element_type=jnp.float32)
            + jnp.dot(cl_ref[...], th, preferred_element_type=jnp.float32))
    zvt_ref[0, :, :NT] = zcol.T


def _tc_call(rms_vel, a_mat, ch_mat, cl_mat):
    b, _, nt, nx = rms_vel.shape
    return pl.pallas_call(
        _tc_body,
        grid=(b, nx // NXB),
        in_specs=[
            pl.BlockSpec((1, 1, NT, NXB), lambda i, j: (i, 0, 0, j)),
            pl.BlockSpec((NT, NT), lambda i, j: (0, 0)),
            pl.BlockSpec((NT, NT), lambda i, j: (0, 0)),
            pl.BlockSpec((NT, NT), lambda i, j: (0, 0)),
        ],
        out_specs=[
            pl.BlockSpec((1, 1, NT, NXB), lambda i, j: (i, 0, 0, j)),
            pl.BlockSpec((1, NXB, 2 * NT), lambda i, j: (i, j, 0)),
        ],
        out_shape=[
            jax.ShapeDtypeStruct((b, 1, nt, nx), jnp.float32),
            jax.ShapeDtypeStruct((b, nx, 2 * nt), jnp.float32),
        ],
    )(rms_vel, a_mat, ch_mat, cl_mat)


def _search_chunk(zv_buf, o_buf, out_hbm, row):
    qfs, los, his = [], [], []
    for g in range(NGRP):
        kq = jnp.minimum(lax.iota(jnp.int32, NL) + (g * NL), NZ - 1)
        qfs.append(kq.astype(jnp.float32) * DZ)
        q100 = kq * 1000
        los.append(jnp.maximum(q100 // 302 - 2, -1))
        his.append(jnp.minimum(q100 // 58 + 3, NT))

    def trace_body(t, carry):
        ts = jnp.full((NL,), t, jnp.int32)
        for g in range(NGRP):
            qf = qfs[g]
            lo = los[g]
            hi = his[g]
            for _ in range(_BS_ITERS[g]):
                mid = jnp.maximum((lo + hi) >> 1, 0)
                zm = plsc.load_gather(zv_buf, [ts, mid])
                pred = zm < qf
                lo = jnp.where(pred, mid, lo)
                hi = jnp.where(pred, hi, mid)
            idx1 = jnp.minimum(hi, NT - 1)
            idx0 = jnp.maximum(idx1 - 1, 0)
            z0 = plsc.load_gather(zv_buf, [ts, idx0])
            z1 = plsc.load_gather(zv_buf, [ts, idx1])
            v0 = plsc.load_gather(zv_buf, [ts, idx0 + NT])
            v1 = plsc.load_gather(zv_buf, [ts, idx1 + NT])
            denom = jnp.maximum(z1 - z0, EPS)
            w = jnp.clip((qf - z0) / denom, 0.0, 1.0)
            o_buf[t, pl.ds(g * NL, NL)] = v0 + w * (v1 - v0)
        return carry

    lax.fori_loop(0, CH, trace_body, 0)
    pltpu.sync_copy(o_buf, out_hbm.at[pl.ds(row, CH)])


def _sc_body(zv_hbm, out_hbm, zv0, zv1, o_buf, sem0, sem1):
    ntr = zv_hbm.shape[0]
    per_w = ntr // NW
    nch = per_w // CH
    wid = lax.axis_index("s") * 2 + lax.axis_index("c")
    base = wid * per_w

    def issue(row, buf, sem):
        pltpu.async_copy(zv_hbm.at[pl.ds(row, CH)], buf, sem)

    def drain(buf, sem):
        pltpu.make_async_copy(zv_hbm.at[pl.ds(0, CH)], buf, sem).wait()

    issue(base, zv0, sem0)

    def pair_body(i, carry):
        r0 = base + (2 * i) * CH
        r1 = r0 + CH
        issue(r1, zv1, sem1)
        drain(zv0, sem0)
        _search_chunk(zv0, o_buf, out_hbm, r0)

        @pl.when(i < (nch // 2) - 1)
        def _():
            issue(r1 + CH, zv0, sem0)

        drain(zv1, sem1)
        _search_chunk(zv1, o_buf, out_hbm, r1)
        return carry

    lax.fori_loop(0, nch // 2, pair_body, 0)


def _depth_resample(zvflat):
    ntr = zvflat.shape[0]
    mesh = plsc.VectorSubcoreMesh(core_axis_name="c", subcore_axis_name="s")
    fn = pl.kernel(
        _sc_body,
        mesh=mesh,
        out_type=jax.ShapeDtypeStruct((ntr, NZP), jnp.float32),
        scratch_types=[
            pltpu.VMEM((CH, 2 * NT), jnp.float32),
            pltpu.VMEM((CH, 2 * NT), jnp.float32),
            pltpu.VMEM((CH, NZP), jnp.float32),
            pltpu.SemaphoreType.DMA,
            pltpu.SemaphoreType.DMA,
        ],
        compiler_params=pltpu.CompilerParams(needs_layout_passes=False),
    )
    return fn(zvflat)


def kernel(rms_vel):
    b, _, nt, nx = rms_vel.shape
    a_mat = jnp.asarray(_A_NP)
    ch_mat = jnp.asarray(_CH_NP)
    cl_mat = jnp.asarray(_CL_NP)
    sm, zvt = _tc_call(rms_vel, a_mat, ch_mat, cl_mat)
    depth_flat = _depth_resample(zvt.reshape(b * nx, 2 * nt))
    depth = depth_flat[:, :NZ].reshape(b, nx, NZ)
    depth_vel = jnp.transpose(depth, (0, 2, 1))[:, None]
    return depth_vel, sm
